# baseline (speedup 1.0000x reference)
import jax
import jax.numpy as jnp
from jax.experimental import pallas as pl
from jax.experimental.pallas import tpu as pltpu


def _proj_body(x_ref, w_ref, o_ref):
    o_ref[...] = jnp.dot(x_ref[...], w_ref[...],
                         preferred_element_type=jnp.float32)


def _mega_body(adj_ref, s_ref, b1_ref, w3_ref, b3_ref, wf_ref, bf_ref,
               o_ref, t_scr, *, bm):
    g = pl.program_id(0)
    nb = pl.num_programs(0) // 2
    a = adj_ref[...]

    @pl.when(g < nb)
    def _layer1():
        acc = jnp.dot(a, s_ref[...], preferred_element_type=jnp.float32)
        h = jnp.maximum(acc + b1_ref[...], 0.0)
        t_scr[pl.ds(g * bm, bm), :] = jnp.dot(
            h, w3_ref[...], preferred_element_type=jnp.float32)

    @pl.when(g >= nb)
    def _layer2():
        acc = jnp.dot(a, t_scr[...], preferred_element_type=jnp.float32)
        h = jnp.maximum(acc + b3_ref[...], 0.0)
        y = jnp.dot(h, wf_ref[...], preferred_element_type=jnp.float32)
        o_ref[...] = jnp.tanh(y + bf_ref[...])


@jax.jit
def _in_proj(x, w):
    n = x.shape[0]
    return pl.pallas_call(
        _proj_body,
        out_shape=jax.ShapeDtypeStruct((n, w.shape[1]), jnp.float32),
    )(x, w)


@jax.jit
def _gcn_fused(adj, s1, b1, w3, b3, wf, bf):
    n, f = adj.shape[0], s1.shape[1]
    bm = 200 if n % 200 == 0 else 8
    nb = n // bm
    import functools
    return pl.pallas_call(
        functools.partial(_mega_body, bm=bm),
        grid=(2 * nb,),
        in_specs=[
            pl.BlockSpec((bm, n), lambda g: (g % nb, 0)),
            pl.BlockSpec((n, f), lambda g: (0, 0)),
            pl.BlockSpec((1, f), lambda g: (0, 0)),
            pl.BlockSpec((f, f), lambda g: (0, 0)),
            pl.BlockSpec((1, f), lambda g: (0, 0)),
            pl.BlockSpec((f, f), lambda g: (0, 0)),
            pl.BlockSpec((1, f), lambda g: (0, 0)),
        ],
        out_specs=pl.BlockSpec((bm, f), lambda g: (g % nb, 0)),
        out_shape=jax.ShapeDtypeStruct((n, f), jnp.float32),
        scratch_shapes=[pltpu.VMEM((n, f), jnp.float32)],
        compiler_params=pltpu.CompilerParams(
            dimension_semantics=("arbitrary",)),
    )(adj, s1, b1, w3, b3, wf, bf)


def kernel(x, adj, W1, b1, W3, b3, Wf, bf):
    b1r = b1.reshape(1, -1)
    b3r = b3.reshape(1, -1)
    bfr = bf.reshape(1, -1)
    s1 = _in_proj(x, W1)
    return _gcn_fused(adj, s1, b1r, W3, b3r, Wf, bfr)


# speedup vs baseline: 1.1012x; 1.1012x over previous
import functools

import jax
import jax.numpy as jnp
from jax.experimental import pallas as pl


def _proj_body(x_ref, w_ref, o_ref):
    o_ref[...] = jnp.dot(x_ref[...], w_ref[...],
                         preferred_element_type=jnp.float32)


def _layer_body(adj_ref, s_ref, b_ref, w_ref, b2_ref, o_ref, *, final):
    acc = jnp.dot(adj_ref[...], s_ref[...],
                  preferred_element_type=jnp.float32)
    h = jnp.maximum(acc + b_ref[...], 0.0)
    y = jnp.dot(h, w_ref[...], preferred_element_type=jnp.float32)
    if final:
        y = jnp.tanh(y + b2_ref[...])
    o_ref[...] = y


@functools.partial(jax.jit, static_argnames=("final",))
def _gcn_layer(adj, s, b, w, b2, final):
    n, f = adj.shape[0], s.shape[1]
    bm = 400 if n % 400 == 0 else 8
    return pl.pallas_call(
        functools.partial(_layer_body, final=final),
        grid=(n // bm,),
        in_specs=[
            pl.BlockSpec((bm, n), lambda i: (i, 0)),
            pl.BlockSpec((n, f), lambda i: (0, 0)),
            pl.BlockSpec((1, f), lambda i: (0, 0)),
            pl.BlockSpec((f, f), lambda i: (0, 0)),
            pl.BlockSpec((1, f), lambda i: (0, 0)),
        ],
        out_specs=pl.BlockSpec((bm, f), lambda i: (i, 0)),
        out_shape=jax.ShapeDtypeStruct((n, f), jnp.float32),
    )(adj, s, b, w, b2)


@jax.jit
def _in_proj(x, w):
    n = x.shape[0]
    return pl.pallas_call(
        _proj_body,
        out_shape=jax.ShapeDtypeStruct((n, w.shape[1]), jnp.float32),
    )(x, w)


def kernel(x, adj, W1, b1, W3, b3, Wf, bf):
    b1r = b1.reshape(1, -1)
    b3r = b3.reshape(1, -1)
    bfr = bf.reshape(1, -1)
    s1 = _in_proj(x, W1)
    t = _gcn_layer(adj, s1, b1r, W3, b3r, final=False)
    out = _gcn_layer(adj, t, b3r, Wf, bfr, final=True)
    return out
